# baseline (speedup 1.0000x reference)
import functools

import jax
import jax.numpy as jnp
from jax import lax
from jax.experimental import pallas as pl
from jax.experimental.pallas import tpu as pltpu
from jax.experimental.pallas import tpu_sc as plsc

NC = 2
NS = 16
NW = NC * NS
L = 16

B = 16384
S = 50
D = 32
CB = B // NW
XW = CB * S * 2
NPAIR = CB // 2
GP = 112
NVR = GP // L
NBUF = 4
NOUTER = NPAIR // NBUF


def _tree_sum(vals):
    accs = list(vals[:4])
    for i in range(4, len(vals)):
        accs[i % 4] = accs[i % 4] + vals[i]
    return (accs[0] + accs[1]) + (accs[2] + accs[3])


def _sc_body(x_hbm, table_hbm, w_hbm, b_hbm, out_hbm,
             xv, idx_v, ps_v, out_v, bufs, sems, wv, bv):
    wid = lax.axis_index("s") * NC + lax.axis_index("c")
    base_row = wid * CB

    pltpu.sync_copy(x_hbm.at[pl.ds(wid * XW, XW)], xv)

    iota = lax.iota(jnp.int32, L)
    zeros = jnp.zeros((L,), jnp.int32)

    def extract(j):
        for t in range(NVR):
            p = iota + t * L
            if (t + 1) * L <= 2 * S:
                vals = plsc.load_gather(xv, [200 * j + 2 * p])
                ivals = vals.astype(jnp.int32)
            else:
                mask = p < 2 * S
                pos = jnp.where(mask, 200 * j + 2 * p, 0)
                vals = plsc.load_gather(xv, [pos], mask=mask)
                ivals = jnp.where(mask, vals.astype(jnp.int32), 0)
            idx_v[j, pl.ds(t * L, L)] = ivals

    def start(j, buf, sem):
        pltpu.make_async_copy(table_hbm.at[idx_v.at[j]], buf, sem).start()

    for bi in range(NBUF):
        extract(bi)
        start(bi, bufs[bi], sems[bi])

    pltpu.sync_copy(w_hbm, wv)
    pltpu.sync_copy(b_hbm, bv)
    w_h = [wv[pl.ds(0, L)], wv[pl.ds(L, L)]]
    sb_h = [bv[pl.ds(0, L)] * float(S), bv[pl.ds(L, L)] * float(S)]

    def ps_body(c, carry):
        base = 100 * (c * L + iota) + 1
        accs = [plsc.load_gather(xv, [base + 2 * s]) for s in range(4)]
        for s in range(4, S):
            accs[s % 4] = accs[s % 4] + plsc.load_gather(xv, [base + 2 * s])
        ps_v[pl.ds(c * L, L)] = (accs[0] + accs[1]) + (accs[2] + accs[3])
        return carry

    lax.fori_loop(0, CB // L, ps_body, 0)

    def accum(j, buf):
        ps_vec = ps_v[pl.ds(2 * j, L)]
        for r in range(2):
            row = 2 * j + r
            ps_s = ps_vec[r]
            for h in range(2):
                tot = _tree_sum(
                    [buf[r * S + i, pl.ds(h * L, L)] for i in range(S)])
                out_v[row, pl.ds(h * L, L)] = tot + ps_s * w_h[h] + sb_h[h]

    def body(i, carry):
        for bi in range(NBUF):
            j = i * NBUF + bi
            pltpu.make_async_copy(
                table_hbm.at[idx_v.at[j]], bufs[bi], sems[bi]).wait()
            accum(j, bufs[bi])

            @pl.when(i < NOUTER - 1)
            def _():
                extract(j + NBUF)
                start(j + NBUF, bufs[bi], sems[bi])
        return carry

    lax.fori_loop(0, NOUTER, body, 0)

    pltpu.sync_copy(out_v, out_hbm.at[pl.ds(base_row, CB)])


@functools.lru_cache(maxsize=1)
def _make_sc_kernel():
    @functools.partial(
        pl.kernel,
        out_type=jax.ShapeDtypeStruct((B, D), jnp.float32),
        mesh=plsc.VectorSubcoreMesh(core_axis_name="c", subcore_axis_name="s",
                                    num_cores=NC, num_subcores=NS),
        compiler_params=pltpu.CompilerParams(use_tc_tiling_on_sc=False, needs_layout_passes=False),
        scratch_types=dict(
            xv=pltpu.VMEM((XW,), jnp.float32),
            idx_v=pltpu.VMEM((NPAIR, GP), jnp.int32),
            ps_v=pltpu.VMEM((CB + L,), jnp.float32),
            out_v=pltpu.VMEM((CB, D), jnp.float32),
            bufs=[pltpu.VMEM((GP, D), jnp.float32) for _ in range(NBUF)],
            sems=[pltpu.SemaphoreType.DMA for _ in range(NBUF)],
            wv=pltpu.VMEM((D,), jnp.float32),
            bv=pltpu.VMEM((D,), jnp.float32),
        ),
    )
    def _sc_kernel(x_hbm, table_hbm, w_hbm, b_hbm, out_hbm,
                   xv, idx_v, ps_v, out_v, bufs, sems, wv, bv):
        _sc_body(x_hbm, table_hbm, w_hbm, b_hbm, out_hbm,
                 xv, idx_v, ps_v, out_v, bufs, sems, wv, bv)

    return _sc_kernel


def kernel(x, table, W, b):
    return _make_sc_kernel()(x.reshape(-1), table, W.reshape(-1), b)


# speedup vs baseline: 1.3946x; 1.3946x over previous
import functools

import jax
import jax.numpy as jnp
from jax import lax
from jax.experimental import pallas as pl
from jax.experimental.pallas import tpu as pltpu
from jax.experimental.pallas import tpu_sc as plsc

NC = 2
NS = 16
NW = NC * NS
L = 16

B = 16384
S = 50
D = 32
CB = B // NW
XW = CB * S * 2
NPAIR = CB // 2
GP = 112
GD = 104
NVR = GP // L
NBUF = 4
NOUTER = NPAIR // NBUF


def _tree_sum(vals):
    accs = list(vals[:4])
    for i in range(4, len(vals)):
        accs[i % 4] = accs[i % 4] + vals[i]
    return (accs[0] + accs[1]) + (accs[2] + accs[3])


def _sc_body(x_hbm, table_hbm, w_hbm, b_hbm, out_hbm,
             xv, idx_v, ps_v, out_v, bufs, sems, wv, bv):
    wid = lax.axis_index("s") * NC + lax.axis_index("c")
    base_row = wid * CB

    pltpu.sync_copy(x_hbm.at[wid], xv.at[pl.ds(0, XW)])

    iota = lax.iota(jnp.int32, L)
    perm_e = (2 * iota) & (L - 1)
    perm_o = (2 * iota + 1) & (L - 1)
    sel8 = iota < 8
    fzero = jnp.zeros((L,), jnp.float32)

    def extract(j):
        pair = 200 * j
        acc_a = fzero
        acc_b = fzero
        for t in range(NVR):
            a = xv[pl.ds(pair + 32 * t, L)]
            b = xv[pl.ds(pair + 32 * t + L, L)]
            ve = jnp.where(sel8, a[perm_e], b[perm_e])
            vo = jnp.where(sel8, a[perm_o], b[perm_o])
            p = iota + t * L
            if (t + 1) * L <= 2 * S:
                ivals = ve.astype(jnp.int32)
            else:
                ivals = jnp.where(p < 2 * S, ve.astype(jnp.int32), 0)
            idx_v[j, pl.ds(t * L, L)] = ivals
            if (t + 1) * L <= S:
                acc_a = acc_a + vo
            elif t * L >= S and (t + 1) * L <= 2 * S:
                acc_b = acc_b + vo
            else:
                acc_a = acc_a + jnp.where(p < S, vo, fzero)
                acc_b = acc_b + jnp.where((p >= S) & (p < 2 * S), vo, fzero)
        for k in (8, 4, 2, 1):
            rot = (iota + k) & (L - 1)
            acc_a = acc_a + acc_a[rot]
            acc_b = acc_b + acc_b[rot]
        ps_v[j, :] = jnp.where(iota < 1, acc_a, acc_b)

    def start(j, buf, sem):
        pltpu.make_async_copy(
            table_hbm.at[idx_v.at[j, pl.ds(0, GD)]], buf, sem).start()

    for bi in range(NBUF):
        extract(bi)
        start(bi, bufs[bi], sems[bi])

    pltpu.sync_copy(w_hbm, wv)
    pltpu.sync_copy(b_hbm, bv)
    w_h = [wv[pl.ds(0, L)], wv[pl.ds(L, L)]]
    sb_h = [bv[pl.ds(0, L)] * float(S), bv[pl.ds(L, L)] * float(S)]

    def accum(j, buf):
        ps_vec = ps_v[j, :]
        for r in range(2):
            row = 2 * j + r
            ps_s = ps_vec[r]
            for h in range(2):
                tot = _tree_sum(
                    [buf[r * S + i, pl.ds(h * L, L)] for i in range(S)])
                out_v[row, pl.ds(h * L, L)] = tot + ps_s * w_h[h] + sb_h[h]

    def body(i, carry):
        for bi in range(NBUF):
            j = i * NBUF + bi
            pltpu.make_async_copy(
                table_hbm.at[idx_v.at[j, pl.ds(0, GD)]],
                bufs[bi], sems[bi]).wait()
            accum(j, bufs[bi])

            @pl.when(i < NOUTER - 1)
            def _():
                extract(j + NBUF)
                start(j + NBUF, bufs[bi], sems[bi])
        return carry

    lax.fori_loop(0, NOUTER, body, 0)

    pltpu.sync_copy(out_v, out_hbm.at[pl.ds(base_row, CB)])


@functools.lru_cache(maxsize=1)
def _make_sc_kernel():
    @functools.partial(
        pl.kernel,
        out_type=jax.ShapeDtypeStruct((B, D), jnp.float32),
        mesh=plsc.VectorSubcoreMesh(core_axis_name="c", subcore_axis_name="s",
                                    num_cores=NC, num_subcores=NS),
        compiler_params=pltpu.CompilerParams(use_tc_tiling_on_sc=False),
        scratch_types=dict(
            xv=pltpu.VMEM((XW + 2 * L,), jnp.float32),
            idx_v=pltpu.VMEM((NPAIR, GP), jnp.int32),
            ps_v=pltpu.VMEM((NPAIR, L), jnp.float32),
            out_v=pltpu.VMEM((CB, D), jnp.float32),
            bufs=[pltpu.VMEM((GD, D), jnp.float32) for _ in range(NBUF)],
            sems=[pltpu.SemaphoreType.DMA for _ in range(NBUF)],
            wv=pltpu.VMEM((D,), jnp.float32),
            bv=pltpu.VMEM((D,), jnp.float32),
        ),
    )
    def _sc_kernel(x_hbm, table_hbm, w_hbm, b_hbm, out_hbm,
                   xv, idx_v, ps_v, out_v, bufs, sems, wv, bv):
        _sc_body(x_hbm, table_hbm, w_hbm, b_hbm, out_hbm,
                 xv, idx_v, ps_v, out_v, bufs, sems, wv, bv)

    return _sc_kernel


def kernel(x, table, W, b):
    return _make_sc_kernel()(x.reshape(NW, XW), table, W.reshape(-1), b)


# speedup vs baseline: 11.0023x; 7.8893x over previous
import functools

import jax
import jax.numpy as jnp
from jax import lax
from jax.experimental import pallas as pl
from jax.experimental.pallas import tpu as pltpu
from jax.experimental.pallas import tpu_sc as plsc

NC = 2
NS = 16
NW = NC * NS
L = 16

B = 16384
S = 50
D = 32
CB = B // NW
NPAIR = CB // 2
G = 2 * S
NBUF = 4
NOUTER = NPAIR // NBUF


def _tree_sum(vals):
    accs = list(vals[:4])
    for i in range(4, len(vals)):
        accs[i % 4] = accs[i % 4] + vals[i]
    return (accs[0] + accs[1]) + (accs[2] + accs[3])


def _sc_body(idx_hbm, props_hbm, table_hbm, w_hbm, b_hbm, out_hbm,
             idx_v, props_v, ps_v, out_v, bufs, wv, bv, sems):
    wid = lax.axis_index("s") * NC + lax.axis_index("c")

    pltpu.sync_copy(idx_hbm.at[wid], idx_v)

    def start(j, buf, sem):
        pltpu.make_async_copy(table_hbm.at[idx_v.at[j]], buf, sem).start()

    for bi in range(NBUF):
        start(bi, bufs[bi], sems[bi])

    pltpu.sync_copy(props_hbm.at[wid], props_v)
    pltpu.sync_copy(w_hbm, wv)
    pltpu.sync_copy(b_hbm, bv)

    iota = lax.iota(jnp.int32, L)
    w_h = [wv[pl.ds(0, L)], wv[pl.ds(L, L)]]
    sb_h = [bv[pl.ds(0, L)] * float(S), bv[pl.ds(L, L)] * float(S)]
    tail_mask = iota >= 4 * L - S
    fzero = jnp.zeros((L,), jnp.float32)

    def ps_body(j, carry):
        sums = []
        for r in range(2):
            row = 2 * j + r
            acc = (props_v[row, pl.ds(0, L)] + props_v[row, pl.ds(L, L)]
                   + props_v[row, pl.ds(2 * L, L)])
            acc = acc + jnp.where(tail_mask,
                                  props_v[row, pl.ds(S - L, L)], fzero)
            for k in (8, 4, 2, 1):
                acc = acc + acc[(iota + k) & (L - 1)]
            sums.append(acc)
        ps_v[j, :] = jnp.where(iota < 1, sums[0], sums[1])
        return carry

    lax.fori_loop(0, NPAIR, ps_body, 0)

    def accum(j, buf):
        ps_vec = ps_v[j, :]
        for r in range(2):
            row = 2 * j + r
            ps_s = ps_vec[r]
            for h in range(2):
                tot = _tree_sum(
                    [buf[r * S + i, pl.ds(h * L, L)] for i in range(S)])
                out_v[row, pl.ds(h * L, L)] = tot + ps_s * w_h[h] + sb_h[h]

    def body(i, carry):
        for bi in range(NBUF):
            j = i * NBUF + bi
            pltpu.make_async_copy(
                table_hbm.at[idx_v.at[j]], bufs[bi], sems[bi]).wait()
            accum(j, bufs[bi])

            @pl.when(i < NOUTER - 1)
            def _():
                start(j + NBUF, bufs[bi], sems[bi])
        return carry

    lax.fori_loop(0, NOUTER, body, 0)

    pltpu.sync_copy(out_v, out_hbm.at[pl.ds(wid * CB, CB)])


@functools.lru_cache(maxsize=1)
def _make_sc_kernel():
    @functools.partial(
        pl.kernel,
        out_type=jax.ShapeDtypeStruct((B, D), jnp.float32),
        mesh=plsc.VectorSubcoreMesh(core_axis_name="c", subcore_axis_name="s",
                                    num_cores=NC, num_subcores=NS),
        compiler_params=pltpu.CompilerParams(use_tc_tiling_on_sc=False),
        scratch_types=dict(
            idx_v=pltpu.VMEM((NPAIR, G), jnp.int32),
            props_v=pltpu.VMEM((CB, S), jnp.float32),
            ps_v=pltpu.VMEM((NPAIR, L), jnp.float32),
            out_v=pltpu.VMEM((CB, D), jnp.float32),
            bufs=[pltpu.VMEM((G, D), jnp.float32) for _ in range(NBUF)],
            wv=pltpu.VMEM((D,), jnp.float32),
            bv=pltpu.VMEM((D,), jnp.float32),
            sems=[pltpu.SemaphoreType.DMA for _ in range(NBUF)],
        ),
    )
    def _sc_kernel(idx_hbm, props_hbm, table_hbm, w_hbm, b_hbm, out_hbm,
                   idx_v, props_v, ps_v, out_v, bufs, wv, bv, sems):
        _sc_body(idx_hbm, props_hbm, table_hbm, w_hbm, b_hbm, out_hbm,
                 idx_v, props_v, ps_v, out_v, bufs, wv, bv, sems)

    return _sc_kernel


def kernel(x, table, W, b):
    idx = x[..., 0].astype(jnp.int32).reshape(NW, NPAIR, G)
    props = x[..., 1].reshape(NW, CB, S)
    w = W[:, 0]
    return _make_sc_kernel()(idx, props, table, w, b)


# speedup vs baseline: 11.2344x; 1.0211x over previous
import functools

import jax
import jax.numpy as jnp
from jax import lax
from jax.experimental import pallas as pl
from jax.experimental.pallas import tpu as pltpu
from jax.experimental.pallas import tpu_sc as plsc

NC = 2
NS = 16
NW = NC * NS
L = 16

B = 16384
S = 50
D = 32
CB = B // NW
NPAIR = CB // 2
G = 2 * S
NBUF = 4
NOUTER = NPAIR // NBUF


def _tree_sum(vals):
    accs = list(vals[:4])
    for i in range(4, len(vals)):
        accs[i % 4] = accs[i % 4] + vals[i]
    return (accs[0] + accs[1]) + (accs[2] + accs[3])


def _sc_body(idx_hbm, props_hbm, table_hbm, w_hbm, b_hbm, out_hbm,
             idx_f, idx_v, props_v, ps_v, out_v, bufs, wv, bv, sems):
    wid = lax.axis_index("s") * NC + lax.axis_index("c")
    nwk = CB * S

    pltpu.sync_copy(idx_hbm.at[pl.ds(wid * nwk, nwk)], idx_f)

    def repack(j, carry):
        base = G * j
        for t in range(6):
            idx_v[j, pl.ds(t * L, L)] = idx_f[pl.ds(base + t * L, L)]
        idx_v[j, pl.ds(G - L, L)] = idx_f[pl.ds(base + G - L, L)]
        return carry

    lax.fori_loop(0, NPAIR, repack, 0)

    def start(j, buf, sem):
        pltpu.make_async_copy(table_hbm.at[idx_v.at[j]], buf, sem).start()

    for bi in range(NBUF):
        start(bi, bufs[bi], sems[bi])

    pltpu.sync_copy(props_hbm.at[pl.ds(wid * nwk, nwk)], props_v)
    pltpu.sync_copy(w_hbm, wv)
    pltpu.sync_copy(b_hbm, bv)

    iota = lax.iota(jnp.int32, L)
    w_h = [wv[pl.ds(0, L)], wv[pl.ds(L, L)]]
    sb_h = [bv[pl.ds(0, L)] * float(S), bv[pl.ds(L, L)] * float(S)]
    tail_mask = iota >= 4 * L - S
    fzero = jnp.zeros((L,), jnp.float32)

    def ps_body(j, carry):
        sums = []
        for r in range(2):
            rb = S * (2 * j + r)
            acc = (props_v[pl.ds(rb, L)] + props_v[pl.ds(rb + L, L)]
                   + props_v[pl.ds(rb + 2 * L, L)])
            acc = acc + jnp.where(tail_mask,
                                  props_v[pl.ds(rb + S - L, L)], fzero)
            for k in (8, 4, 2, 1):
                acc = acc + acc[(iota + k) & (L - 1)]
            sums.append(acc)
        ps_v[j, :] = jnp.where(iota < 1, sums[0], sums[1])
        return carry

    lax.fori_loop(0, NPAIR, ps_body, 0)

    def accum(j, buf):
        ps_vec = ps_v[j, :]
        for r in range(2):
            row = 2 * j + r
            ps_s = ps_vec[r]
            for h in range(2):
                tot = _tree_sum(
                    [buf[r * S + i, pl.ds(h * L, L)] for i in range(S)])
                out_v[row, pl.ds(h * L, L)] = tot + ps_s * w_h[h] + sb_h[h]

    def body(i, carry):
        for bi in range(NBUF):
            j = i * NBUF + bi
            pltpu.make_async_copy(
                table_hbm.at[idx_v.at[j]], bufs[bi], sems[bi]).wait()
            accum(j, bufs[bi])

            @pl.when(i < NOUTER - 1)
            def _():
                start(j + NBUF, bufs[bi], sems[bi])
        return carry

    lax.fori_loop(0, NOUTER, body, 0)

    pltpu.sync_copy(out_v, out_hbm.at[pl.ds(wid * CB, CB)])


@functools.lru_cache(maxsize=1)
def _make_sc_kernel():
    @functools.partial(
        pl.kernel,
        out_type=jax.ShapeDtypeStruct((B, D), jnp.float32),
        mesh=plsc.VectorSubcoreMesh(core_axis_name="c", subcore_axis_name="s",
                                    num_cores=NC, num_subcores=NS),
        compiler_params=pltpu.CompilerParams(use_tc_tiling_on_sc=False),
        scratch_types=dict(
            idx_f=pltpu.VMEM((CB * S,), jnp.int32),
            idx_v=pltpu.VMEM((NPAIR, G), jnp.int32),
            props_v=pltpu.VMEM((CB * S,), jnp.float32),
            ps_v=pltpu.VMEM((NPAIR, L), jnp.float32),
            out_v=pltpu.VMEM((CB, D), jnp.float32),
            bufs=[pltpu.VMEM((G, D), jnp.float32) for _ in range(NBUF)],
            wv=pltpu.VMEM((D,), jnp.float32),
            bv=pltpu.VMEM((D,), jnp.float32),
            sems=[pltpu.SemaphoreType.DMA for _ in range(NBUF)],
        ),
    )
    def _sc_kernel(idx_hbm, props_hbm, table_hbm, w_hbm, b_hbm, out_hbm,
                   idx_f, idx_v, props_v, ps_v, out_v, bufs, wv, bv, sems):
        _sc_body(idx_hbm, props_hbm, table_hbm, w_hbm, b_hbm, out_hbm,
                 idx_f, idx_v, props_v, ps_v, out_v, bufs, wv, bv, sems)

    return _sc_kernel


def kernel(x, table, W, b):
    idx = x[..., 0].astype(jnp.int32).reshape(B * S)
    props = x[..., 1].reshape(B * S)
    w = W[:, 0]
    return _make_sc_kernel()(idx, props, table, w, b)
